# baseline (speedup 1.0000x reference)
import functools

import jax
import jax.numpy as jnp
from jax import lax
from jax.experimental import pallas as pl
from jax.experimental.pallas import tpu as pltpu
from jax.experimental.pallas import tpu_sc as plsc

B = 16384
D = 64
W = 2 * D
NC = 2
NS = 16
L = 16
NW = NC * NS
BPW = B // NW
CHUNK = 128
NCHUNK = BPW // CHUNK
GPC = CHUNK // L


def _sc_body(users_hbm, pos_hbm, neg_hbm, utab_hbm, itab_hbm,
             diff_hbm, sq_hbm,
             uidx, pidx, nidx, urid, prid, nrid,
             ub, pb, nb, diff_v, sq_v, sem):
    wid = lax.axis_index("s") * NC + lax.axis_index("c")
    base = wid * BPW

    pltpu.sync_copy(users_hbm.at[pl.ds(base, BPW)], uidx)
    pltpu.sync_copy(pos_hbm.at[pl.ds(base, BPW)], pidx)
    pltpu.sync_copy(neg_hbm.at[pl.ds(base, BPW)], nidx)

    def shift_body(k, _):
        s = pl.ds(k * L, L)
        urid[s] = lax.shift_right_logical(uidx[s], 1)
        prid[s] = lax.shift_right_logical(pidx[s], 1)
        nrid[s] = lax.shift_right_logical(nidx[s], 1)
        return 0
    lax.fori_loop(0, BPW // L, shift_body, 0)

    def fire(c, sel):
        s = pl.ds(c * CHUNK, CHUNK)
        return (
            pltpu.async_copy(utab_hbm.at[urid.at[s]], ub.at[sel], sem),
            pltpu.async_copy(itab_hbm.at[prid.at[s]], pb.at[sel], sem),
            pltpu.async_copy(itab_hbm.at[nrid.at[s]], nb.at[sel], sem),
        )

    lane = lax.iota(jnp.int32, L)
    zero = jnp.zeros((L,), jnp.float32)

    inflight = fire(0, 0)
    sq_acc = zero
    for c in range(NCHUNK):
        sel = c % 2
        for cp in inflight:
            cp.wait()
        if c + 1 < NCHUNK:
            inflight = fire(c + 1, 1 - sel)

        ubc, pbc, nbc = ub.at[sel], pb.at[sel], nb.at[sel]

        def group_body(g, sq_a, c=c, ubc=ubc, pbc=pbc, nbc=nbc):
            rows = g * L + lane
            s16 = pl.ds(c * CHUNK + g * L, L)
            ucol0 = (uidx[s16] & 1) * D
            pcol0 = (pidx[s16] & 1) * D
            ncol0 = (nidx[s16] & 1) * D
            score = zero
            for d in range(D):
                u = plsc.load_gather(ubc, [rows, ucol0 + d])
                p = plsc.load_gather(pbc, [rows, pcol0 + d])
                n = plsc.load_gather(nbc, [rows, ncol0 + d])
                score = score + u * (p - n)
                sq_a = sq_a + (u * u + p * p + n * n)
            diff_v[s16] = score
            return sq_a

        sq_acc = lax.fori_loop(0, GPC, group_body, sq_acc)

    sq_v[...] = sq_acc
    pltpu.sync_copy(diff_v, diff_hbm.at[pl.ds(base, BPW)])
    pltpu.sync_copy(sq_v, sq_hbm.at[pl.ds(wid * L, L)])


def _loss_body(diff_ref, sq_ref, out_ref):
    x = diff_ref[...]
    ls = jnp.minimum(x, 0.0) - jnp.log1p(jnp.exp(-jnp.abs(x)))
    loss = -jnp.sum(ls) / B
    reg = jnp.sum(sq_ref[...]) / B
    out_ref[...] = jnp.reshape(loss + 1e-5 * reg, (1, 1))


@jax.jit
def kernel(users, pos_items, neg_items, user_table, item_table):
    nu = user_table.shape[0]
    ni = item_table.shape[0]
    mesh = plsc.VectorSubcoreMesh(core_axis_name="c", subcore_axis_name="s")
    sc_fn = pl.kernel(
        _sc_body,
        out_type=[
            jax.ShapeDtypeStruct((B,), jnp.float32),
            jax.ShapeDtypeStruct((NW * L,), jnp.float32),
        ],
        mesh=mesh,
        scratch_types=[
            pltpu.VMEM((BPW,), jnp.int32),
            pltpu.VMEM((BPW,), jnp.int32),
            pltpu.VMEM((BPW,), jnp.int32),
            pltpu.VMEM((BPW,), jnp.int32),
            pltpu.VMEM((BPW,), jnp.int32),
            pltpu.VMEM((BPW,), jnp.int32),
            pltpu.VMEM((2, CHUNK, W), jnp.float32),
            pltpu.VMEM((2, CHUNK, W), jnp.float32),
            pltpu.VMEM((2, CHUNK, W), jnp.float32),
            pltpu.VMEM((BPW,), jnp.float32),
            pltpu.VMEM((L,), jnp.float32),
            pltpu.SemaphoreType.DMA,
        ],
        compiler_params=pltpu.CompilerParams(needs_layout_passes=False),
    )
    diff, sq = sc_fn(
        users.astype(jnp.int32),
        pos_items.astype(jnp.int32),
        neg_items.astype(jnp.int32),
        user_table.reshape(nu // 2, W),
        item_table.reshape(ni // 2, W))

    out = pl.pallas_call(
        _loss_body,
        out_shape=jax.ShapeDtypeStruct((1, 1), jnp.float32),
    )(diff.reshape(128, 128), sq.reshape(4, 128))
    return out[0, 0]


# speedup vs baseline: 2.7690x; 2.7690x over previous
import functools

import jax
import jax.numpy as jnp
from jax import lax
from jax.experimental import pallas as pl
from jax.experimental.pallas import tpu as pltpu
from jax.experimental.pallas import tpu_sc as plsc

B = 16384
D = 64
NC = 2
NS = 16
L = 16
DPC = D // NC
RPT = B // NS
NGROUP = RPT // L
NU = 1000000


def _sc_body(users_hbm, pos_hbm, neg_hbm, utabT_hbm, itabT_hbm,
             diff_hbm, sq_hbm,
             sp, uidx, pidx, nidx, udst, pdst, ndst, diff_v, sq_v, sem):
    cid = lax.axis_index("c")
    sid = lax.axis_index("s")
    rbase = sid * RPT
    dbase = cid * DPC

    pltpu.sync_copy(users_hbm.at[pl.ds(rbase, RPT)], uidx)
    pltpu.sync_copy(pos_hbm.at[pl.ds(rbase, RPT)], pidx)
    pltpu.sync_copy(neg_hbm.at[pl.ds(rbase, RPT)], nidx)

    zero = jnp.zeros((L,), jnp.float32)
    sq_acc = zero
    for d_off in range(DPC):
        d = dbase + d_off

        plsc.subcore_barrier()
        @pl.when(sid == 0)
        def _():
            pltpu.sync_copy(utabT_hbm.at[d], sp)
        plsc.subcore_barrier()
        pltpu.async_copy(sp.at[uidx], udst, sem).wait()
        plsc.subcore_barrier()

        @pl.when(sid == 0)
        def _():
            pltpu.sync_copy(itabT_hbm.at[d], sp)
        plsc.subcore_barrier()
        cp_p = pltpu.async_copy(sp.at[pidx], pdst, sem)
        cp_n = pltpu.async_copy(sp.at[nidx], ndst, sem)
        cp_p.wait()
        cp_n.wait()

        def group_body(g, sqa, first=(d_off == 0)):
            s = pl.ds(g * L, L)
            u = udst[s]
            p = pdst[s]
            n = ndst[s]
            score = u * (p - n) if first else diff_v[s] + u * (p - n)
            diff_v[s] = score
            return sqa + (u * u + p * p + n * n)

        sq_acc = lax.fori_loop(0, NGROUP, group_body, sq_acc)

    sq_v[...] = sq_acc
    pltpu.sync_copy(diff_v, diff_hbm.at[pl.ds(cid * B + rbase, RPT)])
    pltpu.sync_copy(sq_v, sq_hbm.at[pl.ds((cid * NS + sid) * L, L)])


def _loss_body(diff_ref, sq_ref, out_ref):
    x = diff_ref[:128, :] + diff_ref[128:, :]
    ls = jnp.minimum(x, 0.0) - jnp.log1p(jnp.exp(-jnp.abs(x)))
    loss = -jnp.sum(ls) / B
    reg = jnp.sum(sq_ref[...]) / B
    out_ref[...] = jnp.reshape(loss + 1e-5 * reg, (1, 1))


@jax.jit
def kernel(users, pos_items, neg_items, user_table, item_table):
    mesh = plsc.VectorSubcoreMesh(core_axis_name="c", subcore_axis_name="s")
    sc_fn = pl.kernel(
        _sc_body,
        out_type=[
            jax.ShapeDtypeStruct((NC * B,), jnp.float32),
            jax.ShapeDtypeStruct((NC * NS * L,), jnp.float32),
        ],
        mesh=mesh,
        scratch_types=[
            pltpu.VMEM_SHARED((NU,), jnp.float32),
            pltpu.VMEM((RPT,), jnp.int32),
            pltpu.VMEM((RPT,), jnp.int32),
            pltpu.VMEM((RPT,), jnp.int32),
            pltpu.VMEM((RPT,), jnp.float32),
            pltpu.VMEM((RPT,), jnp.float32),
            pltpu.VMEM((RPT,), jnp.float32),
            pltpu.VMEM((RPT,), jnp.float32),
            pltpu.VMEM((L,), jnp.float32),
            pltpu.SemaphoreType.DMA,
        ],
        compiler_params=pltpu.CompilerParams(needs_layout_passes=False),
    )
    diff, sq = sc_fn(
        users.astype(jnp.int32),
        pos_items.astype(jnp.int32),
        neg_items.astype(jnp.int32),
        user_table.T,
        item_table.T)

    out = pl.pallas_call(
        _loss_body,
        out_shape=jax.ShapeDtypeStruct((1, 1), jnp.float32),
    )(diff.reshape(256, 128), sq.reshape(4, 128))
    return out[0, 0]


# speedup vs baseline: 3.6082x; 1.3031x over previous
import functools

import jax
import jax.numpy as jnp
from jax import lax
from jax.experimental import pallas as pl
from jax.experimental.pallas import tpu as pltpu
from jax.experimental.pallas import tpu_sc as plsc

B = 16384
D = 64
NC = 2
NS = 16
L = 16
DPC = D // NC
RPT = B // NS
NGROUP = RPT // L
NU = 1000000
THIRD = 333312
CUT = 3 * THIRD
NTAIL = NU - CUT
SENT = 2147483647


def _sc_body(users_hbm, pos_hbm, neg_hbm, utabT_hbm, itabT_hbm,
             utail_hbm, itail_hbm,
             diff_hbm, sq_hbm,
             sp0, sp1, sp2,
             uidx, pidx, nidx,
             tru0, tru1, tru2, trp0, trp1, trp2, trn0, trn1, trn2,
             udst, pdst, ndst, diff_v, sq_v, utail_v, itail_v,
             s0, s1, s2, gsem):
    cid = lax.axis_index("c")
    sid = lax.axis_index("s")
    rbase = sid * RPT
    dbase = cid * DPC
    ssems = (s0, s1, s2)
    sps = (sp0, sp1, sp2)
    trus = (tru0, tru1, tru2)
    trps = (trp0, trp1, trp2)
    trns = (trn0, trn1, trn2)

    pltpu.sync_copy(users_hbm.at[pl.ds(rbase, RPT)], uidx)
    pltpu.sync_copy(pos_hbm.at[pl.ds(rbase, RPT)], pidx)
    pltpu.sync_copy(neg_hbm.at[pl.ds(rbase, RPT)], nidx)
    pltpu.sync_copy(utail_hbm, utail_v)
    pltpu.sync_copy(itail_hbm, itail_v)

    def prep_body(k, _):
        s = pl.ds(k * L, L)
        uv, pv, nv = uidx[s], pidx[s], nidx[s]
        for src, trs in ((uv, trus), (pv, trps), (nv, trns)):
            for piece in range(3):
                lo = piece * THIRD
                m = (src >= lo) & (src < lo + THIRD)
                trs[piece][s] = jnp.where(m, src - lo, SENT)
        diff_v[s] = jnp.zeros((L,), jnp.float32)
        return 0
    lax.fori_loop(0, NGROUP, prep_body, 0)

    def fire(tab, d, piece):
        return pltpu.async_copy(
            tab.at[d, pl.ds(piece * THIRD, THIRD)], sps[piece], ssems[piece])

    def drain(slot):
        pltpu.make_async_copy(
            utabT_hbm.at[0, pl.ds(0, THIRD)], sps[slot], ssems[slot]).wait()

    @pl.when(sid == 0)
    def _():
        for piece in range(3):
            fire(utabT_hbm, dbase, piece)

    zero = jnp.zeros((L,), jnp.float32)

    def dim_body(d_off, sq_acc):
        d = dbase + d_off

        for j in range(3):
            @pl.when(sid == 0)
            def _(j=j):
                drain(j)
            plsc.subcore_barrier()
            pltpu.async_copy(
                sps[j].at[plsc.Indices(trus[j], ignored_value=SENT)],
                udst, gsem).wait()
            plsc.subcore_barrier()

            @pl.when(sid == 0)
            def _(j=j):
                fire(itabT_hbm, d, j)

        for j in range(3):
            @pl.when(sid == 0)
            def _(j=j):
                drain(j)
            plsc.subcore_barrier()
            cp_p = pltpu.async_copy(
                sps[j].at[plsc.Indices(trps[j], ignored_value=SENT)],
                pdst, gsem)
            cp_n = pltpu.async_copy(
                sps[j].at[plsc.Indices(trns[j], ignored_value=SENT)],
                ndst, gsem)
            cp_p.wait()
            cp_n.wait()

            if j == 2:
                d64 = d * NTAIL

                def group_body(g, sqa):
                    s = pl.ds(g * L, L)
                    u, p, n = udst[s], pdst[s], ndst[s]
                    ivu, ivp, ivn = uidx[s], pidx[s], nidx[s]
                    tu = plsc.load_gather(
                        utail_v, [d64 + jnp.maximum(ivu - CUT, 0)])
                    tp = plsc.load_gather(
                        itail_v, [d64 + jnp.maximum(ivp - CUT, 0)])
                    tn = plsc.load_gather(
                        itail_v, [d64 + jnp.maximum(ivn - CUT, 0)])
                    u = jnp.where(ivu >= CUT, tu, u)
                    p = jnp.where(ivp >= CUT, tp, p)
                    n = jnp.where(ivn >= CUT, tn, n)
                    diff_v[s] = diff_v[s] + u * (p - n)
                    return sqa + (u * u + p * p + n * n)

                sq_acc = lax.fori_loop(0, NGROUP, group_body, sq_acc)

            plsc.subcore_barrier()

            @pl.when((sid == 0) & (d_off + 1 < DPC))
            def _(j=j):
                fire(utabT_hbm, d + 1, j)

        return sq_acc

    sq_acc = lax.fori_loop(0, DPC, dim_body, zero)

    sq_v[...] = sq_acc
    pltpu.sync_copy(diff_v, diff_hbm.at[pl.ds(cid * B + rbase, RPT)])
    pltpu.sync_copy(sq_v, sq_hbm.at[pl.ds((cid * NS + sid) * L, L)])


def _loss_body(diff_ref, sq_ref, out_ref):
    x = diff_ref[:128, :] + diff_ref[128:, :]
    ls = jnp.minimum(x, 0.0) - jnp.log1p(jnp.exp(-jnp.abs(x)))
    loss = -jnp.sum(ls) / B
    reg = jnp.sum(sq_ref[...]) / B
    out_ref[...] = jnp.reshape(loss + 1e-5 * reg, (1, 1))


@jax.jit
def kernel(users, pos_items, neg_items, user_table, item_table):
    mesh = plsc.VectorSubcoreMesh(core_axis_name="c", subcore_axis_name="s")
    sc_fn = pl.kernel(
        _sc_body,
        out_type=[
            jax.ShapeDtypeStruct((NC * B,), jnp.float32),
            jax.ShapeDtypeStruct((NC * NS * L,), jnp.float32),
        ],
        mesh=mesh,
        scratch_types=[
            pltpu.VMEM_SHARED((THIRD,), jnp.float32),
            pltpu.VMEM_SHARED((THIRD,), jnp.float32),
            pltpu.VMEM_SHARED((THIRD,), jnp.float32),
            pltpu.VMEM((RPT,), jnp.int32),
            pltpu.VMEM((RPT,), jnp.int32),
            pltpu.VMEM((RPT,), jnp.int32),
            pltpu.VMEM((RPT,), jnp.int32),
            pltpu.VMEM((RPT,), jnp.int32),
            pltpu.VMEM((RPT,), jnp.int32),
            pltpu.VMEM((RPT,), jnp.int32),
            pltpu.VMEM((RPT,), jnp.int32),
            pltpu.VMEM((RPT,), jnp.int32),
            pltpu.VMEM((RPT,), jnp.int32),
            pltpu.VMEM((RPT,), jnp.int32),
            pltpu.VMEM((RPT,), jnp.int32),

            pltpu.VMEM((RPT,), jnp.float32),
            pltpu.VMEM((RPT,), jnp.float32),
            pltpu.VMEM((RPT,), jnp.float32),
            pltpu.VMEM((RPT,), jnp.float32),
            pltpu.VMEM((L,), jnp.float32),
            pltpu.VMEM((D * NTAIL,), jnp.float32),
            pltpu.VMEM((D * NTAIL,), jnp.float32),
            pltpu.SemaphoreType.DMA,
            pltpu.SemaphoreType.DMA,
            pltpu.SemaphoreType.DMA,
            pltpu.SemaphoreType.DMA,
        ],
        compiler_params=pltpu.CompilerParams(needs_layout_passes=False),
    )
    utail = user_table.T[:, CUT:].reshape(D * NTAIL)
    itail = item_table.T[:, CUT:].reshape(D * NTAIL)
    diff, sq = sc_fn(
        users.astype(jnp.int32),
        pos_items.astype(jnp.int32),
        neg_items.astype(jnp.int32),
        user_table.T,
        item_table.T,
        utail, itail)

    out = pl.pallas_call(
        _loss_body,
        out_shape=jax.ShapeDtypeStruct((1, 1), jnp.float32),
    )(diff.reshape(256, 128), sq.reshape(4, 128))
    return out[0, 0]


# speedup vs baseline: 3.6144x; 1.0017x over previous
import functools

import jax
import jax.numpy as jnp
from jax import lax
from jax.experimental import pallas as pl
from jax.experimental.pallas import tpu as pltpu
from jax.experimental.pallas import tpu_sc as plsc

B = 16384
D = 64
NC = 2
NS = 16
L = 16
DPC = D // NC
RPT = B // NS
NGROUP = RPT // L
NU = 1000000
THIRD = 333312
CUT = 3 * THIRD
NTAIL = NU - CUT
SENT = 2147483647


def _sc_body(users_hbm, pos_hbm, neg_hbm, utabT_hbm, itabT_hbm,
             utail_hbm, itail_hbm,
             diff_hbm, sq_hbm,
             sp0, sp1, sp2,
             uidx, pidx, nidx,
             tru0, tru1, tru2, trp0, trp1, trp2, trn0, trn1, trn2,
             udst, pdst, ndst, diff_v, sq_v, utail_v, itail_v,
             s0, s1, s2, gsem):
    cid = lax.axis_index("c")
    sid = lax.axis_index("s")
    rbase = sid * RPT
    dbase = cid * DPC
    ssems = (s0, s1, s2)
    sps = (sp0, sp1, sp2)
    trus = (tru0, tru1, tru2)
    trps = (trp0, trp1, trp2)
    trns = (trn0, trn1, trn2)

    pltpu.sync_copy(users_hbm.at[pl.ds(rbase, RPT)], uidx)
    pltpu.sync_copy(pos_hbm.at[pl.ds(rbase, RPT)], pidx)
    pltpu.sync_copy(neg_hbm.at[pl.ds(rbase, RPT)], nidx)
    pltpu.sync_copy(utail_hbm, utail_v)
    pltpu.sync_copy(itail_hbm, itail_v)

    def prep_body(k, _):
        s = pl.ds(k * L, L)
        uv, pv, nv = uidx[s], pidx[s], nidx[s]
        for src, trs in ((uv, trus), (pv, trps), (nv, trns)):
            for piece in range(3):
                lo = piece * THIRD
                m = (src >= lo) & (src < lo + THIRD)
                trs[piece][s] = jnp.where(m, src - lo, SENT)
        diff_v[s] = jnp.zeros((L,), jnp.float32)
        return 0
    lax.fori_loop(0, NGROUP, prep_body, 0)

    def fire(tab, d, piece):
        half = THIRD // 2
        for h in range(2):
            pltpu.async_copy(
                tab.at[d, pl.ds(piece * THIRD + h * half, half)],
                sps[piece].at[pl.ds(h * half, half)], ssems[piece])

    def drain(slot):
        pltpu.make_async_copy(
            utabT_hbm.at[0, pl.ds(0, THIRD)], sps[slot], ssems[slot]).wait()

    @pl.when(sid == 0)
    def _():
        for piece in range(3):
            fire(utabT_hbm, dbase, piece)

    zero = jnp.zeros((L,), jnp.float32)

    def dim_body(d_off, sq_acc):
        d = dbase + d_off

        for j in range(3):
            @pl.when(sid == 0)
            def _(j=j):
                drain(j)
            plsc.subcore_barrier()
            pltpu.async_copy(
                sps[j].at[plsc.Indices(trus[j], ignored_value=SENT)],
                udst, gsem).wait()
            plsc.subcore_barrier()

            @pl.when(sid == 0)
            def _(j=j):
                fire(itabT_hbm, d, j)

        for j in range(3):
            @pl.when(sid == 0)
            def _(j=j):
                drain(j)
            plsc.subcore_barrier()
            cp_p = pltpu.async_copy(
                sps[j].at[plsc.Indices(trps[j], ignored_value=SENT)],
                pdst, gsem)
            cp_n = pltpu.async_copy(
                sps[j].at[plsc.Indices(trns[j], ignored_value=SENT)],
                ndst, gsem)
            cp_p.wait()
            cp_n.wait()

            if j == 2:
                d64 = d * NTAIL

                def group_body(g, sqa):
                    s = pl.ds(g * L, L)
                    u, p, n = udst[s], pdst[s], ndst[s]
                    ivu, ivp, ivn = uidx[s], pidx[s], nidx[s]
                    tu = plsc.load_gather(
                        utail_v, [d64 + jnp.maximum(ivu - CUT, 0)])
                    tp = plsc.load_gather(
                        itail_v, [d64 + jnp.maximum(ivp - CUT, 0)])
                    tn = plsc.load_gather(
                        itail_v, [d64 + jnp.maximum(ivn - CUT, 0)])
                    u = jnp.where(ivu >= CUT, tu, u)
                    p = jnp.where(ivp >= CUT, tp, p)
                    n = jnp.where(ivn >= CUT, tn, n)
                    diff_v[s] = diff_v[s] + u * (p - n)
                    return sqa + (u * u + p * p + n * n)

                sq_acc = lax.fori_loop(0, NGROUP, group_body, sq_acc)

            plsc.subcore_barrier()

            @pl.when((sid == 0) & (d_off + 1 < DPC))
            def _(j=j):
                fire(utabT_hbm, d + 1, j)

        return sq_acc

    sq_acc = lax.fori_loop(0, DPC, dim_body, zero)

    sq_v[...] = sq_acc
    pltpu.sync_copy(diff_v, diff_hbm.at[pl.ds(cid * B + rbase, RPT)])
    pltpu.sync_copy(sq_v, sq_hbm.at[pl.ds((cid * NS + sid) * L, L)])


def _loss_body(diff_ref, sq_ref, out_ref):
    x = diff_ref[:128, :] + diff_ref[128:, :]
    ls = jnp.minimum(x, 0.0) - jnp.log1p(jnp.exp(-jnp.abs(x)))
    loss = -jnp.sum(ls) / B
    reg = jnp.sum(sq_ref[...]) / B
    out_ref[...] = jnp.reshape(loss + 1e-5 * reg, (1, 1))


@jax.jit
def kernel(users, pos_items, neg_items, user_table, item_table):
    mesh = plsc.VectorSubcoreMesh(core_axis_name="c", subcore_axis_name="s")
    sc_fn = pl.kernel(
        _sc_body,
        out_type=[
            jax.ShapeDtypeStruct((NC * B,), jnp.float32),
            jax.ShapeDtypeStruct((NC * NS * L,), jnp.float32),
        ],
        mesh=mesh,
        scratch_types=[
            pltpu.VMEM_SHARED((THIRD,), jnp.float32),
            pltpu.VMEM_SHARED((THIRD,), jnp.float32),
            pltpu.VMEM_SHARED((THIRD,), jnp.float32),
            pltpu.VMEM((RPT,), jnp.int32),
            pltpu.VMEM((RPT,), jnp.int32),
            pltpu.VMEM((RPT,), jnp.int32),
            pltpu.VMEM((RPT,), jnp.int32),
            pltpu.VMEM((RPT,), jnp.int32),
            pltpu.VMEM((RPT,), jnp.int32),
            pltpu.VMEM((RPT,), jnp.int32),
            pltpu.VMEM((RPT,), jnp.int32),
            pltpu.VMEM((RPT,), jnp.int32),
            pltpu.VMEM((RPT,), jnp.int32),
            pltpu.VMEM((RPT,), jnp.int32),
            pltpu.VMEM((RPT,), jnp.int32),

            pltpu.VMEM((RPT,), jnp.float32),
            pltpu.VMEM((RPT,), jnp.float32),
            pltpu.VMEM((RPT,), jnp.float32),
            pltpu.VMEM((RPT,), jnp.float32),
            pltpu.VMEM((L,), jnp.float32),
            pltpu.VMEM((D * NTAIL,), jnp.float32),
            pltpu.VMEM((D * NTAIL,), jnp.float32),
            pltpu.SemaphoreType.DMA,
            pltpu.SemaphoreType.DMA,
            pltpu.SemaphoreType.DMA,
            pltpu.SemaphoreType.DMA,
        ],
        compiler_params=pltpu.CompilerParams(needs_layout_passes=False),
    )
    utail = user_table.T[:, CUT:].reshape(D * NTAIL)
    itail = item_table.T[:, CUT:].reshape(D * NTAIL)
    diff, sq = sc_fn(
        users.astype(jnp.int32),
        pos_items.astype(jnp.int32),
        neg_items.astype(jnp.int32),
        user_table.T,
        item_table.T,
        utail, itail)

    out = pl.pallas_call(
        _loss_body,
        out_shape=jax.ShapeDtypeStruct((1, 1), jnp.float32),
    )(diff.reshape(256, 128), sq.reshape(4, 128))
    return out[0, 0]
